# Initial kernel scaffold; baseline (speedup 1.0000x reference)
import jax
import jax.numpy as jnp
from jax.experimental import pallas as pl


def kernel(x, edge_index, batch, W1, b1, g1, be1, W2, b2, g2, be2, Wl, bl):
    raise NotImplementedError("write your pallas kernel here")


# speedup vs baseline: 9.7023x; 9.7023x over previous
import functools

import jax
import jax.numpy as jnp
from jax import lax
from jax.experimental import pallas as pl
from jax.experimental.pallas import tpu as pltpu
import jax.experimental.pallas.tpu_sc as plsc

N = 10000
E = 320000
D = 128
DOUT = 64
G = 64
EPS = 1e-5

NC = 2
NS = 16
NW = NC * NS
K = 128
NBLK = -(-E // (K * NW))
EPAD = NBLK * K * NW
RPT = 624
TAIL = N - NS * RPT

BROW = 1000
GRID = N // BROW

_MESH = plsc.VectorSubcoreMesh(core_axis_name="c", subcore_axis_name="s")


@functools.partial(
    pl.kernel,
    out_type=[jax.ShapeDtypeStruct((N, D), jnp.float32),
              jax.ShapeDtypeStruct((N, D), jnp.float32)],
    mesh=_MESH,
    scratch_types=[pltpu.VMEM((K,), jnp.int32),
                   pltpu.VMEM((K, D), jnp.float32),
                   pltpu.VMEM_SHARED((N + 8, D), jnp.float32)],
)
def _sc_degree(dst_hbm, zeros_hbm, ones_hbm, out0, out1, idx_v, ones_v, acc_sh):
    cid = lax.axis_index("c")
    sid = lax.axis_index("s")
    pltpu.sync_copy(ones_hbm, ones_v)
    pltpu.sync_copy(zeros_hbm.at[pl.ds(0, RPT)], acc_sh.at[pl.ds(sid * RPT, RPT)])

    @pl.when(sid == NS - 1)
    def _():
        pltpu.sync_copy(zeros_hbm.at[pl.ds(0, TAIL)],
                        acc_sh.at[pl.ds(NS * RPT, TAIL)])

    plsc.subcore_barrier()
    row0 = (cid * NS + sid) * NBLK

    def body(j, carry):
        pltpu.sync_copy(dst_hbm.at[row0 + j], idx_v)
        pltpu.sync_copy(ones_v, acc_sh.at[idx_v], add=True)
        return carry

    lax.fori_loop(0, NBLK, body, 0)
    plsc.subcore_barrier()

    @pl.when(cid == 0)
    def _():
        pltpu.sync_copy(acc_sh.at[pl.ds(sid * RPT, RPT)],
                        out0.at[pl.ds(sid * RPT, RPT)])

        @pl.when(sid == NS - 1)
        def _():
            pltpu.sync_copy(acc_sh.at[pl.ds(NS * RPT, TAIL)],
                            out0.at[pl.ds(NS * RPT, TAIL)])

    @pl.when(cid == 1)
    def _():
        pltpu.sync_copy(acc_sh.at[pl.ds(sid * RPT, RPT)],
                        out1.at[pl.ds(sid * RPT, RPT)])

        @pl.when(sid == NS - 1)
        def _():
            pltpu.sync_copy(acc_sh.at[pl.ds(NS * RPT, TAIL)],
                            out1.at[pl.ds(NS * RPT, TAIL)])


@functools.partial(
    pl.kernel,
    out_type=[jax.ShapeDtypeStruct((N, D), jnp.float32),
              jax.ShapeDtypeStruct((N, D), jnp.float32)],
    mesh=_MESH,
    scratch_types=[pltpu.VMEM((K,), jnp.int32),
                   pltpu.VMEM((K,), jnp.int32),
                   pltpu.VMEM((K, D), jnp.float32),
                   pltpu.VMEM_SHARED((N + 8, D), jnp.float32),
                   pltpu.SemaphoreType.DMA],
)
def _sc_scatter(src_hbm, dst_hbm, h_hbm, zeros_hbm, out0, out1,
                si_v, di_v, rows_v, acc_sh, sem):
    cid = lax.axis_index("c")
    sid = lax.axis_index("s")
    pltpu.sync_copy(zeros_hbm.at[pl.ds(0, RPT)], acc_sh.at[pl.ds(sid * RPT, RPT)])

    @pl.when(sid == NS - 1)
    def _():
        pltpu.sync_copy(zeros_hbm.at[pl.ds(0, TAIL)],
                        acc_sh.at[pl.ds(NS * RPT, TAIL)])

    plsc.subcore_barrier()
    row0 = (cid * NS + sid) * NBLK

    def body(j, carry):
        pltpu.sync_copy(src_hbm.at[row0 + j], si_v)
        pltpu.sync_copy(dst_hbm.at[row0 + j], di_v)
        pltpu.async_copy(h_hbm.at[si_v], rows_v, sem).wait()
        pltpu.sync_copy(rows_v, acc_sh.at[di_v], add=True)
        return carry

    lax.fori_loop(0, NBLK, body, 0)
    plsc.subcore_barrier()

    @pl.when(cid == 0)
    def _():
        pltpu.sync_copy(acc_sh.at[pl.ds(sid * RPT, RPT)],
                        out0.at[pl.ds(sid * RPT, RPT)])

        @pl.when(sid == NS - 1)
        def _():
            pltpu.sync_copy(acc_sh.at[pl.ds(NS * RPT, TAIL)],
                            out0.at[pl.ds(NS * RPT, TAIL)])

    @pl.when(cid == 1)
    def _():
        pltpu.sync_copy(acc_sh.at[pl.ds(sid * RPT, RPT)],
                        out1.at[pl.ds(sid * RPT, RPT)])

        @pl.when(sid == NS - 1)
        def _():
            pltpu.sync_copy(acc_sh.at[pl.ds(NS * RPT, TAIL)],
                            out1.at[pl.ds(NS * RPT, TAIL)])


def _tc_prep_body(x_ref, w_ref, p0_ref, p1_ref, h_ref, dinv_ref):
    deg = 1.0 + p0_ref[:, 0:1] + p1_ref[:, 0:1]
    dinv = lax.rsqrt(deg)
    h = jnp.dot(x_ref[...], w_ref[...], preferred_element_type=jnp.float32)
    h_ref[...] = h * dinv
    dinv_ref[...] = dinv


_tc_prep = pl.pallas_call(
    _tc_prep_body,
    grid=(GRID,),
    in_specs=[pl.BlockSpec((BROW, D), lambda i: (i, 0)),
              pl.BlockSpec((D, D), lambda i: (0, 0)),
              pl.BlockSpec((BROW, D), lambda i: (i, 0)),
              pl.BlockSpec((BROW, D), lambda i: (i, 0))],
    out_specs=[pl.BlockSpec((BROW, D), lambda i: (i, 0)),
               pl.BlockSpec((BROW, 1), lambda i: (i, 0))],
    out_shape=[jax.ShapeDtypeStruct((N, D), jnp.float32),
               jax.ShapeDtypeStruct((N, 1), jnp.float32)],
)


def _tc_comb_body(a0_ref, a1_ref, hp_ref, dv_ref, b_ref, t_ref, s_ref):
    i = pl.program_id(0)
    t = dv_ref[...] * (a0_ref[...] + a1_ref[...] + hp_ref[...]) + b_ref[...]
    t_ref[...] = t

    @pl.when(i == 0)
    def _():
        s_ref[...] = jnp.zeros_like(s_ref)

    s_ref[0:1, :] += jnp.sum(t, axis=0, keepdims=True)
    s_ref[1:2, :] += jnp.sum(t * t, axis=0, keepdims=True)


_tc_comb = pl.pallas_call(
    _tc_comb_body,
    grid=(GRID,),
    in_specs=[pl.BlockSpec((BROW, D), lambda i: (i, 0)),
              pl.BlockSpec((BROW, D), lambda i: (i, 0)),
              pl.BlockSpec((BROW, D), lambda i: (i, 0)),
              pl.BlockSpec((BROW, 1), lambda i: (i, 0)),
              pl.BlockSpec((1, D), lambda i: (0, 0))],
    out_specs=[pl.BlockSpec((BROW, D), lambda i: (i, 0)),
               pl.BlockSpec((8, D), lambda i: (0, 0))],
    out_shape=[jax.ShapeDtypeStruct((N, D), jnp.float32),
               jax.ShapeDtypeStruct((8, D), jnp.float32)],
)


def _tc_apply_body(t_ref, s_ref, g_ref, be_ref, w_ref, dv_ref, o_ref):
    m = s_ref[0:1, :] * (1.0 / N)
    var = s_ref[1:2, :] * (1.0 / N) - m * m
    xn = (t_ref[...] - m) * lax.rsqrt(var + EPS) * g_ref[...] + be_ref[...]
    h = jnp.maximum(xn, 0.0)
    o_ref[...] = jnp.dot(h, w_ref[...], preferred_element_type=jnp.float32) * dv_ref[...]


_tc_apply = pl.pallas_call(
    _tc_apply_body,
    grid=(GRID,),
    in_specs=[pl.BlockSpec((BROW, D), lambda i: (i, 0)),
              pl.BlockSpec((8, D), lambda i: (0, 0)),
              pl.BlockSpec((1, D), lambda i: (0, 0)),
              pl.BlockSpec((1, D), lambda i: (0, 0)),
              pl.BlockSpec((D, D), lambda i: (0, 0)),
              pl.BlockSpec((BROW, 1), lambda i: (i, 0))],
    out_specs=pl.BlockSpec((BROW, D), lambda i: (i, 0)),
    out_shape=jax.ShapeDtypeStruct((N, D), jnp.float32),
)


def _tc_finish_body(t_ref, s_ref, g_ref, be_ref, b3_ref, wl_ref, bl_ref,
                    o_ref, pool_acc, cnt_acc):
    i = pl.program_id(0)
    m = s_ref[0:1, :] * (1.0 / N)
    var = s_ref[1:2, :] * (1.0 / N) - m * m
    h = jnp.maximum(
        (t_ref[...] - m) * lax.rsqrt(var + EPS) * g_ref[...] + be_ref[...], 0.0)
    ids = b3_ref[0, 0, :]
    oh = (ids[None, :] == lax.broadcasted_iota(jnp.int32, (G, BROW), 0)
          ).astype(jnp.float32)

    @pl.when(i == 0)
    def _():
        pool_acc[...] = jnp.zeros_like(pool_acc)
        cnt_acc[...] = jnp.zeros_like(cnt_acc)

    pool_acc[...] += jnp.dot(oh, h, preferred_element_type=jnp.float32)
    cnt_acc[...] = cnt_acc[...] + jnp.sum(oh, axis=1, keepdims=True)

    @pl.when(i == GRID - 1)
    def _():
        p = pool_acc[...] / jnp.maximum(cnt_acc[...], 1.0)
        o_ref[...] = jnp.dot(p, wl_ref[...], preferred_element_type=jnp.float32) + bl_ref[...]


_tc_finish = pl.pallas_call(
    _tc_finish_body,
    grid=(GRID,),
    in_specs=[pl.BlockSpec((BROW, D), lambda i: (i, 0)),
              pl.BlockSpec((8, D), lambda i: (0, 0)),
              pl.BlockSpec((1, D), lambda i: (0, 0)),
              pl.BlockSpec((1, D), lambda i: (0, 0)),
              pl.BlockSpec((1, 1, BROW), lambda i: (i, 0, 0)),
              pl.BlockSpec((D, DOUT), lambda i: (0, 0)),
              pl.BlockSpec((1, DOUT), lambda i: (0, 0))],
    out_specs=pl.BlockSpec((G, DOUT), lambda i: (0, 0)),
    out_shape=jax.ShapeDtypeStruct((G, DOUT), jnp.float32),
    scratch_shapes=[pltpu.VMEM((G, D), jnp.float32),
                    pltpu.VMEM((G, D), jnp.float32)],
)


def kernel(x, edge_index, batch, W1, b1, g1, be1, W2, b2, g2, be2, Wl, bl):
    pad = EPAD - E
    src = jnp.concatenate([edge_index[0], jnp.zeros((pad,), jnp.int32)])
    dst = jnp.concatenate([edge_index[1], jnp.full((pad,), N, jnp.int32)])
    src = src.reshape(NW * NBLK, K)
    dst = dst.reshape(NW * NBLK, K)
    z128 = jnp.zeros((RPT, D), jnp.float32)
    o128 = jnp.ones((K, D), jnp.float32)

    d0, d1 = _sc_degree(dst, z128, o128)
    h1p, dinv = _tc_prep(x, W1, d0, d1)
    a0, a1 = _sc_scatter(src, dst, h1p, z128)
    t1, s1 = _tc_comb(a0, a1, h1p, dinv, b1.reshape(1, D))
    h2p = _tc_apply(t1, s1, g1.reshape(1, D), be1.reshape(1, D), W2, dinv)
    c0, c1 = _sc_scatter(src, dst, h2p, z128)
    t2, s2 = _tc_comb(c0, c1, h2p, dinv, b2.reshape(1, D))
    return _tc_finish(t2, s2, g2.reshape(1, D), be2.reshape(1, D),
                      batch.reshape(GRID, 1, BROW), Wl, bl.reshape(1, DOUT))
